# Initial kernel scaffold; baseline (speedup 1.0000x reference)
import jax
import jax.numpy as jnp
from jax.experimental import pallas as pl


def kernel(x, edge_index, edge_weight, W, bias):
    raise NotImplementedError("write your pallas kernel here")


# speedup vs baseline: 3.4978x; 3.4978x over previous
import functools

import jax
import jax.numpy as jnp
from jax import lax
from jax.experimental import pallas as pl
from jax.experimental.pallas import tpu as pltpu
from jax.experimental.pallas import tpu_sc as plsc

NC = 2
NS = 16
NW = NC * NS
CHUNK = 128
LANES = 16


def _spmm_partials(v, src, dst, w, zeros, n_chunks, n, d):
    rows_per_sub = n // NS
    per_w = n_chunks * CHUNK
    mesh = plsc.VectorSubcoreMesh(core_axis_name="c", subcore_axis_name="s")

    @functools.partial(
        pl.kernel,
        out_type=jax.ShapeDtypeStruct((NC, n, d), jnp.float32),
        mesh=mesh,
        scratch_types=[
            pltpu.VMEM((CHUNK,), jnp.int32),
            pltpu.VMEM((CHUNK,), jnp.int32),
            pltpu.VMEM((CHUNK,), jnp.float32),
            pltpu.VMEM((CHUNK, d), jnp.float32),
            pltpu.VMEM_SHARED((n, d), jnp.float32),
            pltpu.SemaphoreType.DMA,
        ],
    )
    def k(v_hbm, src_hbm, dst_hbm, w_hbm, z_hbm, out_hbm, si, di, wv, rows, acc, sem):
        cid = lax.axis_index("c")
        sid = lax.axis_index("s")
        wid = cid * NS + sid
        row0 = sid * rows_per_sub
        pltpu.sync_copy(z_hbm.at[pl.ds(row0, rows_per_sub)],
                        acc.at[pl.ds(row0, rows_per_sub)])
        plsc.subcore_barrier()
        base = wid * per_w

        def chunk_body(c, carry):
            off = base + c * CHUNK
            pltpu.sync_copy(src_hbm.at[pl.ds(off, CHUNK)], si)
            pltpu.sync_copy(dst_hbm.at[pl.ds(off, CHUNK)], di)
            pltpu.sync_copy(w_hbm.at[pl.ds(off, CHUNK)], wv)
            pltpu.async_copy(v_hbm.at[si], rows, sem).wait()

            def group_body(g, gcarry):
                wvec = wv[pl.ds(g * LANES, LANES)]
                for i in range(LANES):
                    wgt = wvec[i]
                    e = g * LANES + i
                    for j in range(d // LANES):
                        sl = pl.ds(j * LANES, LANES)
                        rows[e, sl] = rows[e, sl] * wgt
                return gcarry

            lax.fori_loop(0, CHUNK // LANES, group_body, 0)
            pltpu.sync_copy(rows, acc.at[di], add=True)
            return carry

        lax.fori_loop(0, n_chunks, chunk_body, 0)
        plsc.subcore_barrier()
        pltpu.sync_copy(acc.at[pl.ds(row0, rows_per_sub)],
                        out_hbm.at[cid, pl.ds(row0, rows_per_sub)])

    return k(v, src, dst, w, zeros)


def _combine_tc(x, p):
    def body(x_ref, p_ref, y_ref):
        y_ref[...] = x_ref[...] + 2.0 * (p_ref[0] + p_ref[1])

    return pl.pallas_call(
        body, out_shape=jax.ShapeDtypeStruct(x.shape, jnp.float32))(x, p)


def _matmul_tc(q, w_mat, bias2d):
    def body(q_ref, w_ref, b_ref, o_ref):
        s = q_ref[0] + q_ref[1]
        o_ref[...] = jnp.dot(s, w_ref[...],
                             preferred_element_type=jnp.float32) + b_ref[...]

    n, d = q.shape[1], q.shape[2]
    return pl.pallas_call(
        body, out_shape=jax.ShapeDtypeStruct((n, d), jnp.float32))(q, w_mat, bias2d)


def kernel(x, edge_index, edge_weight, W, bias):
    n, d = x.shape
    src = edge_index[0]
    dst = edge_index[1]
    e = src.shape[0]
    n_chunks = -(-e // (NW * CHUNK))
    e_pad = NW * CHUNK * n_chunks
    pad = e_pad - e
    if pad:
        src = jnp.pad(src, (0, pad))
        dst = jnp.pad(dst, (0, pad))
        edge_weight = jnp.pad(edge_weight, (0, pad))
    n_pad = -(-n // (NS * 8)) * (NS * 8)
    x_pad = jnp.pad(x, ((0, n_pad - n), (0, 0))) if n_pad != n else x
    zeros = jnp.zeros_like(x_pad)
    p = _spmm_partials(x_pad, src, dst, edge_weight, zeros, n_chunks, n_pad, d)
    y = _combine_tc(x_pad, p)
    q = _spmm_partials(y, src, dst, edge_weight, zeros, n_chunks, n_pad, d)
    return _matmul_tc(q, W, bias.reshape(1, d))[:n]
